# Initial kernel scaffold; baseline (speedup 1.0000x reference)
import jax
import jax.numpy as jnp
from jax.experimental import pallas as pl


def kernel(emb, W_enc, b_enc, W_pvnd, b_pvnd, codebook):
    raise NotImplementedError("write your pallas kernel here")


# speedup vs baseline: 2.2981x; 2.2981x over previous
import functools

import jax
import jax.numpy as jnp
from jax import lax
from jax.experimental import pallas as pl

EPS = 1e-08
PI = 0.95
RSV_DIM = 1
K = 8192
D = 64
C_IN = 96
BETA = 0.25

B = 8
HW = 1024
N = B * HW
TB = 256
NTB = N // TB


def _pv_const():
    Pi = PI * jnp.ones((D - RSV_DIM,), dtype=jnp.float32)
    return (jnp.concatenate([jnp.ones((1,), jnp.float32), jnp.cumprod(Pi)])
            * jnp.concatenate([1.0 - Pi, jnp.ones((1,), jnp.float32)]))


def _front_body(emb_ref, we_ref, be_ref, wp_ref, bp_ref, g_ref, pv_ref,
                lat_ref, kld_ref):
    x = emb_ref[0]
    feat = lax.dot_general(x, we_ref[...], (((0,), (1,)), ((), ())),
                           preferred_element_type=jnp.float32)
    feat = feat + be_ref[...]
    pvnd = lax.dot_general(x, wp_ref[...], (((0,), (1,)), ((), ())),
                           preferred_element_type=jnp.float32)
    pvnd = pvnd + bp_ref[...]

    beta = jax.nn.sigmoid(jnp.clip(pvnd[:, RSV_DIM:], -5.0, 5.0))
    ones1 = jnp.ones((HW, 1), jnp.float32)
    cp = jnp.concatenate([ones1, beta], axis=1)
    for sh in (1, 2, 4, 8, 16, 32):
        shifted = jnp.concatenate(
            [jnp.ones((HW, sh), jnp.float32), cp[:, :D - sh]], axis=1)
        cp = cp * shifted
    qv = cp * jnp.concatenate([1.0 - beta, ones1], axis=1)

    z = qv + g_ref[0]
    m = jnp.max(z, axis=1, keepdims=True)
    is_max = (z == m).astype(jnp.float32)
    lane = lax.broadcasted_iota(jnp.int32, (D, D), 0)
    lane2 = lax.broadcasted_iota(jnp.int32, (D, D), 1)
    strict_tri = (lane < lane2).astype(jnp.float32)
    excl = lax.dot_general(is_max, strict_tri, (((1,), (0,)), ((), ())),
                           preferred_element_type=jnp.float32)
    mask = (excl == 0.0).astype(jnp.float32)

    lat_ref[0] = feat * mask

    logf = jnp.log(qv / pv_ref[...] + EPS)
    kld_ref[0] = jnp.sum(qv * logf, axis=1)[None, :]


def _dist_body(flat_ref, cn_ref, cb_ref, out_ref, mse_ref):
    f = flat_ref[0]
    fn = jnp.sum(f * f, axis=1, keepdims=True)
    mm = lax.dot_general(f, cb_ref[...], (((1,), (1,)), ((), ())),
                         preferred_element_type=jnp.float32)
    dist = (fn + cn_ref[...]) - 2.0 * mm
    dmin = jnp.min(dist, axis=1, keepdims=True)
    col = lax.broadcasted_iota(jnp.int32, (TB, K), 1)
    idx = jnp.min(jnp.where(dist == dmin, col, K), axis=1)
    oh = (col == idx[:, None]).astype(jnp.float32)
    q = lax.dot_general(oh, cb_ref[...], (((1,), (0,)), ((), ())),
                        preferred_element_type=jnp.float32)
    d = q - f
    out_ref[0] = (f + d).T
    mse_ref[0] = jnp.sum(d * d, axis=1)[None, :]


@jax.jit
def kernel(emb, W_enc, b_enc, W_pvnd, b_pvnd, codebook):
    emb_r = emb.reshape(B, C_IN, HW)
    u = jax.random.uniform(jax.random.key(42), (B, 32, 32, D),
                           dtype=jnp.float32, minval=1e-10, maxval=1.0)
    g = (-jnp.log(-jnp.log(u))).reshape(B, HW, D)
    pv = _pv_const().reshape(1, D)

    lat, kld_rows = pl.pallas_call(
        _front_body,
        grid=(B,),
        in_specs=[
            pl.BlockSpec((1, C_IN, HW), lambda b: (b, 0, 0)),
            pl.BlockSpec((D, C_IN), lambda b: (0, 0)),
            pl.BlockSpec((1, D), lambda b: (0, 0)),
            pl.BlockSpec((D, C_IN), lambda b: (0, 0)),
            pl.BlockSpec((1, D), lambda b: (0, 0)),
            pl.BlockSpec((1, HW, D), lambda b: (b, 0, 0)),
            pl.BlockSpec((1, D), lambda b: (0, 0)),
        ],
        out_specs=[
            pl.BlockSpec((1, HW, D), lambda b: (b, 0, 0)),
            pl.BlockSpec((1, 1, HW), lambda b: (b, 0, 0)),
        ],
        out_shape=[
            jax.ShapeDtypeStruct((B, HW, D), jnp.float32),
            jax.ShapeDtypeStruct((B, 1, HW), jnp.float32),
        ],
    )(emb_r, W_enc, b_enc.reshape(1, D), W_pvnd, b_pvnd.reshape(1, D), g, pv)

    flat = lat.reshape(N, D)
    cnorm = jnp.sum(codebook ** 2, axis=1).reshape(1, K)

    outT, mse_rows = pl.pallas_call(
        _dist_body,
        grid=(NTB,),
        in_specs=[
            pl.BlockSpec((1, TB, D), lambda i: (i, 0, 0)),
            pl.BlockSpec((1, K), lambda i: (0, 0)),
            pl.BlockSpec((K, D), lambda i: (0, 0)),
        ],
        out_specs=[
            pl.BlockSpec((1, D, TB), lambda i: (i // 4, 0, i % 4)),
            pl.BlockSpec((1, 1, TB), lambda i: (i, 0, 0)),
        ],
        out_shape=[
            jax.ShapeDtypeStruct((B, D, HW), jnp.float32),
            jax.ShapeDtypeStruct((NTB, 1, TB), jnp.float32),
        ],
    )(flat.reshape(NTB, TB, D), cnorm, codebook)

    mse = jnp.sum(mse_rows) / (N * D)
    kld = jnp.sum(kld_rows) / N
    vq_loss = (mse + kld) * BETA + mse
    return (outT.reshape(B, D, 32, 32), vq_loss)


# speedup vs baseline: 3.0815x; 1.3409x over previous
import functools

import jax
import jax.numpy as jnp
from jax import lax
from jax.experimental import pallas as pl
from jax.experimental.pallas import tpu as pltpu
from jax.experimental.pallas import tpu_sc as plsc

EPS = 1e-08
PI = 0.95
RSV_DIM = 1
K = 8192
D = 64
C_IN = 96
BETA = 0.25

B = 8
HW = 1024
N = B * HW
TB = 256
NTB = N // TB


def _pv_const():
    Pi = PI * jnp.ones((D - RSV_DIM,), dtype=jnp.float32)
    return (jnp.concatenate([jnp.ones((1,), jnp.float32), jnp.cumprod(Pi)])
            * jnp.concatenate([1.0 - Pi, jnp.ones((1,), jnp.float32)]))


def _front_body(emb_ref, we_ref, be_ref, wp_ref, bp_ref, g_ref, pv_ref,
                lat_ref, kld_ref):
    x = emb_ref[0]
    feat = lax.dot_general(x, we_ref[...], (((0,), (1,)), ((), ())),
                           preferred_element_type=jnp.float32)
    feat = feat + be_ref[...]
    pvnd = lax.dot_general(x, wp_ref[...], (((0,), (1,)), ((), ())),
                           preferred_element_type=jnp.float32)
    pvnd = pvnd + bp_ref[...]

    beta = jax.nn.sigmoid(jnp.clip(pvnd[:, RSV_DIM:], -5.0, 5.0))
    ones1 = jnp.ones((HW, 1), jnp.float32)
    cp = jnp.concatenate([ones1, beta], axis=1)
    for sh in (1, 2, 4, 8, 16, 32):
        shifted = jnp.concatenate(
            [jnp.ones((HW, sh), jnp.float32), cp[:, :D - sh]], axis=1)
        cp = cp * shifted
    qv = cp * jnp.concatenate([1.0 - beta, ones1], axis=1)

    z = qv + g_ref[0]
    m = jnp.max(z, axis=1, keepdims=True)
    is_max = (z == m).astype(jnp.float32)
    lane = lax.broadcasted_iota(jnp.int32, (D, D), 0)
    lane2 = lax.broadcasted_iota(jnp.int32, (D, D), 1)
    strict_tri = (lane < lane2).astype(jnp.float32)
    excl = lax.dot_general(is_max, strict_tri, (((1,), (0,)), ((), ())),
                           preferred_element_type=jnp.float32)
    mask = (excl == 0.0).astype(jnp.float32)

    lat_ref[0] = feat * mask

    logf = jnp.log(qv / pv_ref[...] + EPS)
    kld_ref[0] = jnp.sum(qv * logf, axis=1)[None, :]


def _dist_body(flat_ref, cn_ref, cb_ref, idx_ref):
    f = flat_ref[0]
    fn = jnp.sum(f * f, axis=1, keepdims=True)
    mm = lax.dot_general(f, cb_ref[...], (((1,), (1,)), ((), ())),
                         preferred_element_type=jnp.float32)
    dist = (fn + cn_ref[...]) - 2.0 * mm
    dmin = jnp.min(dist, axis=1, keepdims=True)
    col = lax.broadcasted_iota(jnp.int32, (TB, K), 1)
    idx_ref[0] = jnp.min(jnp.where(dist == dmin, col, K), axis=1)[None, :]


def _combine_body(lat_ref, q_ref, out_ref, mse_ref):
    l = lat_ref[0]
    q = q_ref[0][:, :D]
    d = q - l
    out_ref[0] = (l + d).T
    mse_ref[0] = jnp.sum(d * d, axis=1)[None, :]


_SC_INFO = plsc.get_sparse_core_info()
_NC = _SC_INFO.num_cores
_NS = _SC_INFO.num_subcores
_NW = _NC * _NS
_RPW = N // _NW
_CHUNK = 128
_NCH = _RPW // _CHUNK


def _gather_body(cb_hbm, idx_hbm, out_hbm, idx_v, rows_v, sem):
    wid = lax.axis_index("s") * _NC + lax.axis_index("c")
    pltpu.sync_copy(idx_hbm.at[pl.ds(wid * _NCH, _NCH)], idx_v)
    copies = []
    for j in range(_NCH):
        copies.append(pltpu.async_copy(
            cb_hbm.at[idx_v.at[j]], rows_v.at[pl.ds(j * _CHUNK, _CHUNK)], sem))
    for c in copies:
        c.wait()
    pltpu.sync_copy(rows_v, out_hbm.at[pl.ds(wid * _RPW, _RPW)])


_PD = 128

_sc_gather = functools.partial(
    pl.kernel,
    out_type=jax.ShapeDtypeStruct((N, _PD), jnp.float32),
    mesh=plsc.VectorSubcoreMesh(core_axis_name="c", subcore_axis_name="s"),
    scratch_types=[
        pltpu.VMEM((_NCH, _CHUNK), jnp.int32),
        pltpu.VMEM((_RPW, _PD), jnp.float32),
        pltpu.SemaphoreType.DMA,
    ],
)(_gather_body)


@jax.jit
def kernel(emb, W_enc, b_enc, W_pvnd, b_pvnd, codebook):
    emb_r = emb.reshape(B, C_IN, HW)
    u = jax.random.uniform(jax.random.key(42), (B, 32, 32, D),
                           dtype=jnp.float32, minval=1e-10, maxval=1.0)
    g = (-jnp.log(-jnp.log(u))).reshape(B, HW, D)
    pv = _pv_const().reshape(1, D)

    lat, kld_rows = pl.pallas_call(
        _front_body,
        grid=(B,),
        in_specs=[
            pl.BlockSpec((1, C_IN, HW), lambda b: (b, 0, 0)),
            pl.BlockSpec((D, C_IN), lambda b: (0, 0)),
            pl.BlockSpec((1, D), lambda b: (0, 0)),
            pl.BlockSpec((D, C_IN), lambda b: (0, 0)),
            pl.BlockSpec((1, D), lambda b: (0, 0)),
            pl.BlockSpec((1, HW, D), lambda b: (b, 0, 0)),
            pl.BlockSpec((1, D), lambda b: (0, 0)),
        ],
        out_specs=[
            pl.BlockSpec((1, HW, D), lambda b: (b, 0, 0)),
            pl.BlockSpec((1, 1, HW), lambda b: (b, 0, 0)),
        ],
        out_shape=[
            jax.ShapeDtypeStruct((B, HW, D), jnp.float32),
            jax.ShapeDtypeStruct((B, 1, HW), jnp.float32),
        ],
    )(emb_r, W_enc, b_enc.reshape(1, D), W_pvnd, b_pvnd.reshape(1, D), g, pv)

    flat = lat.reshape(NTB, TB, D)
    cnorm = jnp.sum(codebook ** 2, axis=1).reshape(1, K)

    idx = pl.pallas_call(
        _dist_body,
        grid=(NTB,),
        in_specs=[
            pl.BlockSpec((1, TB, D), lambda i: (i, 0, 0)),
            pl.BlockSpec((1, K), lambda i: (0, 0)),
            pl.BlockSpec((K, D), lambda i: (0, 0)),
        ],
        out_specs=pl.BlockSpec((1, 1, TB), lambda i: (i, 0, 0)),
        out_shape=jax.ShapeDtypeStruct((NTB, 1, TB), jnp.int32),
    )(flat, cnorm, codebook)

    cb_pad = jnp.pad(codebook, ((0, 0), (0, _PD - D)))
    q = _sc_gather(cb_pad, idx.reshape(N // _CHUNK, _CHUNK))

    outT, mse_rows = pl.pallas_call(
        _combine_body,
        grid=(NTB,),
        in_specs=[
            pl.BlockSpec((1, TB, D), lambda i: (i, 0, 0)),
            pl.BlockSpec((1, TB, _PD), lambda i: (i, 0, 0)),
        ],
        out_specs=[
            pl.BlockSpec((1, D, TB), lambda i: (i // 4, 0, i % 4)),
            pl.BlockSpec((1, 1, TB), lambda i: (i, 0, 0)),
        ],
        out_shape=[
            jax.ShapeDtypeStruct((B, D, HW), jnp.float32),
            jax.ShapeDtypeStruct((NTB, 1, TB), jnp.float32),
        ],
    )(flat, q.reshape(NTB, TB, _PD))

    mse = jnp.sum(mse_rows) / (N * D)
    kld = jnp.sum(kld_rows) / N
    vq_loss = (mse + kld) * BETA + mse
    return (outT.reshape(B, D, 32, 32), vq_loss)


# speedup vs baseline: 3.1913x; 1.0356x over previous
import functools

import jax
import jax.numpy as jnp
from jax import lax
from jax.experimental import pallas as pl
from jax.experimental.pallas import tpu as pltpu
from jax.experimental.pallas import tpu_sc as plsc

EPS = 1e-08
PI = 0.95
RSV_DIM = 1
K = 8192
D = 64
C_IN = 96
BETA = 0.25

B = 8
HW = 1024
N = B * HW
TB = 256
NTB = N // TB


def _pv_const():
    Pi = PI * jnp.ones((D - RSV_DIM,), dtype=jnp.float32)
    return (jnp.concatenate([jnp.ones((1,), jnp.float32), jnp.cumprod(Pi)])
            * jnp.concatenate([1.0 - Pi, jnp.ones((1,), jnp.float32)]))


def _front_body(emb_ref, we_ref, be_ref, wp_ref, bp_ref, g_ref, pv_ref,
                lat_ref, kld_ref):
    x = emb_ref[0]
    feat = lax.dot_general(x, we_ref[...], (((0,), (1,)), ((), ())),
                           preferred_element_type=jnp.float32)
    feat = feat + be_ref[...]
    pvnd = lax.dot_general(x, wp_ref[...], (((0,), (1,)), ((), ())),
                           preferred_element_type=jnp.float32)
    pvnd = pvnd + bp_ref[...]

    beta = jax.nn.sigmoid(jnp.clip(pvnd[:, RSV_DIM:], -5.0, 5.0))
    ones1 = jnp.ones((HW, 1), jnp.float32)
    cp = jnp.concatenate([ones1, beta], axis=1)
    for sh in (1, 2, 4, 8, 16, 32):
        shifted = jnp.concatenate(
            [jnp.ones((HW, sh), jnp.float32), cp[:, :D - sh]], axis=1)
        cp = cp * shifted
    qv = cp * jnp.concatenate([1.0 - beta, ones1], axis=1)

    z = qv + g_ref[0]
    m = jnp.max(z, axis=1, keepdims=True)
    is_max = (z == m).astype(jnp.float32)
    lane = lax.broadcasted_iota(jnp.int32, (D, D), 0)
    lane2 = lax.broadcasted_iota(jnp.int32, (D, D), 1)
    strict_tri = (lane < lane2).astype(jnp.float32)
    excl = lax.dot_general(is_max, strict_tri, (((1,), (0,)), ((), ())),
                           preferred_element_type=jnp.float32)
    mask = (excl == 0.0).astype(jnp.float32)

    lat_ref[0] = feat * mask

    logf = jnp.log(qv / pv_ref[...] + EPS)
    kld_ref[0] = jnp.sum(qv * logf, axis=1)[None, :]


def _dist_body(flat_ref, cn_ref, cb_ref, io_ref, idx_ref):
    f = flat_ref[0]
    fn = jnp.sum(f * f, axis=1, keepdims=True)
    mm = lax.dot_general(f, cb_ref[...], (((1,), (1,)), ((), ())),
                         preferred_element_type=jnp.float32)
    dist = (fn + cn_ref[...]) - 2.0 * mm
    dmin = jnp.min(dist, axis=1, keepdims=True)
    idxf = jnp.min(jnp.where(dist == dmin, io_ref[...], jnp.float32(K)),
                   axis=1)
    idx_ref[0] = idxf.astype(jnp.int32)[None, :]


def _combine_body(lat_ref, q_ref, out_ref, mse_ref):
    l = lat_ref[0]
    q = q_ref[0][:, :D]
    d = q - l
    out_ref[0] = (l + d).T
    mse_ref[0] = jnp.sum(d * d, axis=1)[None, :]


_SC_INFO = plsc.get_sparse_core_info()
_NC = _SC_INFO.num_cores
_NS = _SC_INFO.num_subcores
_NW = _NC * _NS
_RPW = N // _NW
_CHUNK = 128
_NCH = _RPW // _CHUNK


def _gather_body(cb_hbm, idx_hbm, out_hbm, idx_v, rows_v, sem):
    wid = lax.axis_index("s") * _NC + lax.axis_index("c")
    pltpu.sync_copy(idx_hbm.at[pl.ds(wid * _NCH, _NCH)], idx_v)
    copies = []
    for j in range(_NCH):
        copies.append(pltpu.async_copy(
            cb_hbm.at[idx_v.at[j]], rows_v.at[pl.ds(j * _CHUNK, _CHUNK)], sem))
    for c in copies:
        c.wait()
    pltpu.sync_copy(rows_v, out_hbm.at[pl.ds(wid * _RPW, _RPW)])


_PD = D

_sc_gather = functools.partial(
    pl.kernel,
    out_type=jax.ShapeDtypeStruct((N, _PD), jnp.float32),
    mesh=plsc.VectorSubcoreMesh(core_axis_name="c", subcore_axis_name="s"),
    compiler_params=pltpu.CompilerParams(use_tc_tiling_on_sc=False),
    scratch_types=[
        pltpu.VMEM((_NCH, _CHUNK), jnp.int32),
        pltpu.VMEM((_RPW, _PD), jnp.float32),
        pltpu.SemaphoreType.DMA,
    ],
)(_gather_body)


@jax.jit
def kernel(emb, W_enc, b_enc, W_pvnd, b_pvnd, codebook):
    emb_r = emb.reshape(B, C_IN, HW)
    u = jax.random.uniform(jax.random.key(42), (B, 32, 32, D),
                           dtype=jnp.float32, minval=1e-10, maxval=1.0)
    g = (-jnp.log(-jnp.log(u))).reshape(B, HW, D)
    pv = _pv_const().reshape(1, D)

    lat, kld_rows = pl.pallas_call(
        _front_body,
        grid=(B,),
        in_specs=[
            pl.BlockSpec((1, C_IN, HW), lambda b: (b, 0, 0)),
            pl.BlockSpec((D, C_IN), lambda b: (0, 0)),
            pl.BlockSpec((1, D), lambda b: (0, 0)),
            pl.BlockSpec((D, C_IN), lambda b: (0, 0)),
            pl.BlockSpec((1, D), lambda b: (0, 0)),
            pl.BlockSpec((1, HW, D), lambda b: (b, 0, 0)),
            pl.BlockSpec((1, D), lambda b: (0, 0)),
        ],
        out_specs=[
            pl.BlockSpec((1, HW, D), lambda b: (b, 0, 0)),
            pl.BlockSpec((1, 1, HW), lambda b: (b, 0, 0)),
        ],
        out_shape=[
            jax.ShapeDtypeStruct((B, HW, D), jnp.float32),
            jax.ShapeDtypeStruct((B, 1, HW), jnp.float32),
        ],
    )(emb_r, W_enc, b_enc.reshape(1, D), W_pvnd, b_pvnd.reshape(1, D), g, pv)

    flat = lat.reshape(NTB, TB, D)
    cnorm = jnp.sum(codebook ** 2, axis=1).reshape(1, K)
    iota_f = jnp.arange(K, dtype=jnp.float32).reshape(1, K)

    idx = pl.pallas_call(
        _dist_body,
        grid=(NTB,),
        in_specs=[
            pl.BlockSpec((1, TB, D), lambda i: (i, 0, 0)),
            pl.BlockSpec((1, K), lambda i: (0, 0)),
            pl.BlockSpec((K, D), lambda i: (0, 0)),
            pl.BlockSpec((1, K), lambda i: (0, 0)),
        ],
        out_specs=pl.BlockSpec((1, 1, TB), lambda i: (i, 0, 0)),
        out_shape=jax.ShapeDtypeStruct((NTB, 1, TB), jnp.int32),
    )(flat, cnorm, codebook, iota_f)

    q = _sc_gather(codebook, idx.reshape(N // _CHUNK, _CHUNK))

    outT, mse_rows = pl.pallas_call(
        _combine_body,
        grid=(NTB,),
        in_specs=[
            pl.BlockSpec((1, TB, D), lambda i: (i, 0, 0)),
            pl.BlockSpec((1, TB, _PD), lambda i: (i, 0, 0)),
        ],
        out_specs=[
            pl.BlockSpec((1, D, TB), lambda i: (i // 4, 0, i % 4)),
            pl.BlockSpec((1, 1, TB), lambda i: (i, 0, 0)),
        ],
        out_shape=[
            jax.ShapeDtypeStruct((B, D, HW), jnp.float32),
            jax.ShapeDtypeStruct((NTB, 1, TB), jnp.float32),
        ],
    )(flat, q.reshape(NTB, TB, _PD))

    mse = jnp.sum(mse_rows) / (N * D)
    kld = jnp.sum(kld_rows) / N
    vq_loss = (mse + kld) * BETA + mse
    return (outT.reshape(B, D, 32, 32), vq_loss)


# speedup vs baseline: 3.4889x; 1.0932x over previous
import functools

import jax
import jax.numpy as jnp
from jax import lax
from jax.experimental import pallas as pl
from jax.experimental.pallas import tpu as pltpu
from jax.experimental.pallas import tpu_sc as plsc

EPS = 1e-08
PI = 0.95
RSV_DIM = 1
K = 8192
D = 64
C_IN = 96
BETA = 0.25

B = 8
HW = 1024
N = B * HW
TB = 512
NTB = N // TB


def _pv_const():
    Pi = PI * jnp.ones((D - RSV_DIM,), dtype=jnp.float32)
    return (jnp.concatenate([jnp.ones((1,), jnp.float32), jnp.cumprod(Pi)])
            * jnp.concatenate([1.0 - Pi, jnp.ones((1,), jnp.float32)]))


def _front_body(emb_ref, we_ref, be_ref, wp_ref, bp_ref, g_ref, pv_ref,
                lat_ref, kld_ref):
    x = emb_ref[0]
    feat = lax.dot_general(x, we_ref[...], (((0,), (1,)), ((), ())),
                           preferred_element_type=jnp.float32)
    feat = feat + be_ref[...]
    pvnd = lax.dot_general(x, wp_ref[...], (((0,), (1,)), ((), ())),
                           preferred_element_type=jnp.float32)
    pvnd = pvnd + bp_ref[...]

    beta = jax.nn.sigmoid(jnp.clip(pvnd[:, RSV_DIM:], -5.0, 5.0))
    ones1 = jnp.ones((HW, 1), jnp.float32)
    cp = jnp.concatenate([ones1, beta], axis=1)
    for sh in (1, 2, 4, 8, 16, 32):
        shifted = jnp.concatenate(
            [jnp.ones((HW, sh), jnp.float32), cp[:, :D - sh]], axis=1)
        cp = cp * shifted
    qv = cp * jnp.concatenate([1.0 - beta, ones1], axis=1)

    z = qv + g_ref[0]
    m = jnp.max(z, axis=1, keepdims=True)
    is_max = (z == m).astype(jnp.float32)
    lane = lax.broadcasted_iota(jnp.int32, (D, D), 0)
    lane2 = lax.broadcasted_iota(jnp.int32, (D, D), 1)
    strict_tri = (lane < lane2).astype(jnp.float32)
    excl = lax.dot_general(is_max, strict_tri, (((1,), (0,)), ((), ())),
                           preferred_element_type=jnp.float32)
    mask = (excl == 0.0).astype(jnp.float32)

    lat_ref[0] = feat * mask

    logf = jnp.log(qv / pv_ref[...] + EPS)
    kld_ref[0] = jnp.sum(qv * logf, axis=1)[None, :]


def _dist_body(flat_ref, cn_ref, cb_ref, io_ref, idx_ref):
    f = flat_ref[0]
    fn = jnp.sum(f * f, axis=1, keepdims=True)
    mm = lax.dot_general(f, cb_ref[...], (((1,), (1,)), ((), ())),
                         preferred_element_type=jnp.float32)
    dist = (fn + cn_ref[...]) - 2.0 * mm
    dmin = jnp.min(dist, axis=1, keepdims=True)
    idxf = jnp.min(jnp.where(dist == dmin, io_ref[...], jnp.float32(K)),
                   axis=1)
    idx_ref[0] = idxf.astype(jnp.int32)[None, :]


def _combine_body(lat_ref, q_ref, out_ref, mse_ref):
    l = lat_ref[0]
    q = q_ref[0][:, :D]
    d = q - l
    out_ref[0] = (l + d).T
    mse_ref[0] = jnp.sum(d * d, axis=1)[None, :]


_SC_INFO = plsc.get_sparse_core_info()
_NC = _SC_INFO.num_cores
_NS = _SC_INFO.num_subcores
_NW = _NC * _NS
_RPW = N // _NW
_CHUNK = 128
_NCH = _RPW // _CHUNK


def _gather_body(cb_hbm, idx_hbm, out_hbm, idx_v, rows_v, sem):
    wid = lax.axis_index("s") * _NC + lax.axis_index("c")
    pltpu.sync_copy(idx_hbm.at[pl.ds(wid * _NCH, _NCH)], idx_v)
    copies = []
    for j in range(_NCH):
        copies.append(pltpu.async_copy(
            cb_hbm.at[idx_v.at[j]], rows_v.at[pl.ds(j * _CHUNK, _CHUNK)], sem))
    for c in copies:
        c.wait()
    pltpu.sync_copy(rows_v, out_hbm.at[pl.ds(wid * _RPW, _RPW)])


_PD = D

_sc_gather = functools.partial(
    pl.kernel,
    out_type=jax.ShapeDtypeStruct((N, _PD), jnp.float32),
    mesh=plsc.VectorSubcoreMesh(core_axis_name="c", subcore_axis_name="s"),
    compiler_params=pltpu.CompilerParams(use_tc_tiling_on_sc=False),
    scratch_types=[
        pltpu.VMEM((_NCH, _CHUNK), jnp.int32),
        pltpu.VMEM((_RPW, _PD), jnp.float32),
        pltpu.SemaphoreType.DMA,
    ],
)(_gather_body)


@jax.jit
def kernel(emb, W_enc, b_enc, W_pvnd, b_pvnd, codebook):
    emb_r = emb.reshape(B, C_IN, HW)
    u = jax.random.uniform(jax.random.key(42), (B, 32, 32, D),
                           dtype=jnp.float32, minval=1e-10, maxval=1.0)
    g = (-jnp.log(-jnp.log(u))).reshape(B, HW, D)
    pv = _pv_const().reshape(1, D)

    lat, kld_rows = pl.pallas_call(
        _front_body,
        grid=(B,),
        in_specs=[
            pl.BlockSpec((1, C_IN, HW), lambda b: (b, 0, 0)),
            pl.BlockSpec((D, C_IN), lambda b: (0, 0)),
            pl.BlockSpec((1, D), lambda b: (0, 0)),
            pl.BlockSpec((D, C_IN), lambda b: (0, 0)),
            pl.BlockSpec((1, D), lambda b: (0, 0)),
            pl.BlockSpec((1, HW, D), lambda b: (b, 0, 0)),
            pl.BlockSpec((1, D), lambda b: (0, 0)),
        ],
        out_specs=[
            pl.BlockSpec((1, HW, D), lambda b: (b, 0, 0)),
            pl.BlockSpec((1, 1, HW), lambda b: (b, 0, 0)),
        ],
        out_shape=[
            jax.ShapeDtypeStruct((B, HW, D), jnp.float32),
            jax.ShapeDtypeStruct((B, 1, HW), jnp.float32),
        ],
    )(emb_r, W_enc, b_enc.reshape(1, D), W_pvnd, b_pvnd.reshape(1, D), g, pv)

    flat = lat.reshape(NTB, TB, D)
    cnorm = jnp.sum(codebook ** 2, axis=1).reshape(1, K)
    iota_f = jnp.arange(K, dtype=jnp.float32).reshape(1, K)

    idx = pl.pallas_call(
        _dist_body,
        grid=(NTB,),
        in_specs=[
            pl.BlockSpec((1, TB, D), lambda i: (i, 0, 0)),
            pl.BlockSpec((1, K), lambda i: (0, 0)),
            pl.BlockSpec((K, D), lambda i: (0, 0)),
            pl.BlockSpec((1, K), lambda i: (0, 0)),
        ],
        out_specs=pl.BlockSpec((1, 1, TB), lambda i: (i, 0, 0)),
        out_shape=jax.ShapeDtypeStruct((NTB, 1, TB), jnp.int32),
        compiler_params=pltpu.CompilerParams(vmem_limit_bytes=100 * 1024 * 1024),
    )(flat, cnorm, codebook, iota_f)

    q = _sc_gather(codebook, idx.reshape(N // _CHUNK, _CHUNK))

    outT, mse_rows = pl.pallas_call(
        _combine_body,
        grid=(NTB,),
        in_specs=[
            pl.BlockSpec((1, TB, D), lambda i: (i, 0, 0)),
            pl.BlockSpec((1, TB, _PD), lambda i: (i, 0, 0)),
        ],
        out_specs=[
            pl.BlockSpec((1, D, TB), lambda i: (i // (HW // TB), 0, i % (HW // TB))),
            pl.BlockSpec((1, 1, TB), lambda i: (i, 0, 0)),
        ],
        out_shape=[
            jax.ShapeDtypeStruct((B, D, HW), jnp.float32),
            jax.ShapeDtypeStruct((NTB, 1, TB), jnp.float32),
        ],
    )(flat, q.reshape(NTB, TB, _PD))

    mse = jnp.sum(mse_rows) / (N * D)
    kld = jnp.sum(kld_rows) / N
    vq_loss = (mse + kld) * BETA + mse
    return (outT.reshape(B, D, 32, 32), vq_loss)


# speedup vs baseline: 3.5881x; 1.0284x over previous
import functools

import jax
import jax.numpy as jnp
from jax import lax
from jax.experimental import pallas as pl
from jax.experimental.pallas import tpu as pltpu
from jax.experimental.pallas import tpu_sc as plsc

EPS = 1e-08
PI = 0.95
RSV_DIM = 1
K = 8192
D = 64
C_IN = 96
BETA = 0.25

B = 8
HW = 1024
N = B * HW
TB = 1024
NTB = N // TB


def _pv_const():
    Pi = PI * jnp.ones((D - RSV_DIM,), dtype=jnp.float32)
    return (jnp.concatenate([jnp.ones((1,), jnp.float32), jnp.cumprod(Pi)])
            * jnp.concatenate([1.0 - Pi, jnp.ones((1,), jnp.float32)]))


def _front_body(emb_ref, we_ref, be_ref, wp_ref, bp_ref, g_ref, pv_ref,
                lat_ref, kld_ref):
    x = emb_ref[0]
    feat = lax.dot_general(x, we_ref[...], (((0,), (1,)), ((), ())),
                           preferred_element_type=jnp.float32)
    feat = feat + be_ref[...]
    pvnd = lax.dot_general(x, wp_ref[...], (((0,), (1,)), ((), ())),
                           preferred_element_type=jnp.float32)
    pvnd = pvnd + bp_ref[...]

    beta = jax.nn.sigmoid(jnp.clip(pvnd[:, RSV_DIM:], -5.0, 5.0))
    ones1 = jnp.ones((HW, 1), jnp.float32)
    cp = jnp.concatenate([ones1, beta], axis=1)
    for sh in (1, 2, 4, 8, 16, 32):
        shifted = jnp.concatenate(
            [jnp.ones((HW, sh), jnp.float32), cp[:, :D - sh]], axis=1)
        cp = cp * shifted
    qv = cp * jnp.concatenate([1.0 - beta, ones1], axis=1)

    z = qv + g_ref[0]
    m = jnp.max(z, axis=1, keepdims=True)
    is_max = (z == m).astype(jnp.float32)
    lane = lax.broadcasted_iota(jnp.int32, (D, D), 0)
    lane2 = lax.broadcasted_iota(jnp.int32, (D, D), 1)
    strict_tri = (lane < lane2).astype(jnp.float32)
    excl = lax.dot_general(is_max, strict_tri, (((1,), (0,)), ((), ())),
                           preferred_element_type=jnp.float32)
    mask = (excl == 0.0).astype(jnp.float32)

    lat_ref[0] = feat * mask

    logf = jnp.log(qv / pv_ref[...] + EPS)
    kld_ref[0] = jnp.sum(qv * logf, axis=1)[None, :]


def _dist_body(flat_ref, cn_ref, cb_ref, io_ref, idx_ref):
    f = flat_ref[0]
    fn = jnp.sum(f * f, axis=1, keepdims=True)
    mm = lax.dot_general(f, cb_ref[...], (((1,), (1,)), ((), ())),
                         preferred_element_type=jnp.float32)
    dist = (fn + cn_ref[...]) - 2.0 * mm
    dmin = jnp.min(dist, axis=1, keepdims=True)
    idxf = jnp.min(jnp.where(dist == dmin, io_ref[...], jnp.float32(K)),
                   axis=1)
    idx_ref[0] = idxf.astype(jnp.int32)[None, :]


def _combine_body(lat_ref, q_ref, out_ref, mse_ref):
    l = lat_ref[0]
    q = q_ref[0][:, :D]
    d = q - l
    out_ref[0] = (l + d).T
    mse_ref[0] = jnp.sum(d * d, axis=1)[None, :]


_SC_INFO = plsc.get_sparse_core_info()
_NC = _SC_INFO.num_cores
_NS = _SC_INFO.num_subcores
_NW = _NC * _NS
_RPW = N // _NW
_CHUNK = 128
_NCH = _RPW // _CHUNK


def _gather_body(cb_hbm, idx_hbm, out_hbm, idx_v, rows_v, sem):
    wid = lax.axis_index("s") * _NC + lax.axis_index("c")
    pltpu.sync_copy(idx_hbm.at[pl.ds(wid * _NCH, _NCH)], idx_v)
    copies = []
    for j in range(_NCH):
        copies.append(pltpu.async_copy(
            cb_hbm.at[idx_v.at[j]], rows_v.at[pl.ds(j * _CHUNK, _CHUNK)], sem))
    for c in copies:
        c.wait()
    pltpu.sync_copy(rows_v, out_hbm.at[pl.ds(wid * _RPW, _RPW)])


_PD = D

_sc_gather = functools.partial(
    pl.kernel,
    out_type=jax.ShapeDtypeStruct((N, _PD), jnp.float32),
    mesh=plsc.VectorSubcoreMesh(core_axis_name="c", subcore_axis_name="s"),
    compiler_params=pltpu.CompilerParams(use_tc_tiling_on_sc=False),
    scratch_types=[
        pltpu.VMEM((_NCH, _CHUNK), jnp.int32),
        pltpu.VMEM((_RPW, _PD), jnp.float32),
        pltpu.SemaphoreType.DMA,
    ],
)(_gather_body)


@jax.jit
def kernel(emb, W_enc, b_enc, W_pvnd, b_pvnd, codebook):
    emb_r = emb.reshape(B, C_IN, HW)
    u = jax.random.uniform(jax.random.key(42), (B, 32, 32, D),
                           dtype=jnp.float32, minval=1e-10, maxval=1.0)
    g = (-jnp.log(-jnp.log(u))).reshape(B, HW, D)
    pv = _pv_const().reshape(1, D)

    lat, kld_rows = pl.pallas_call(
        _front_body,
        grid=(B,),
        in_specs=[
            pl.BlockSpec((1, C_IN, HW), lambda b: (b, 0, 0)),
            pl.BlockSpec((D, C_IN), lambda b: (0, 0)),
            pl.BlockSpec((1, D), lambda b: (0, 0)),
            pl.BlockSpec((D, C_IN), lambda b: (0, 0)),
            pl.BlockSpec((1, D), lambda b: (0, 0)),
            pl.BlockSpec((1, HW, D), lambda b: (b, 0, 0)),
            pl.BlockSpec((1, D), lambda b: (0, 0)),
        ],
        out_specs=[
            pl.BlockSpec((1, HW, D), lambda b: (b, 0, 0)),
            pl.BlockSpec((1, 1, HW), lambda b: (b, 0, 0)),
        ],
        out_shape=[
            jax.ShapeDtypeStruct((B, HW, D), jnp.float32),
            jax.ShapeDtypeStruct((B, 1, HW), jnp.float32),
        ],
    )(emb_r, W_enc, b_enc.reshape(1, D), W_pvnd, b_pvnd.reshape(1, D), g, pv)

    flat = lat.reshape(NTB, TB, D)
    cnorm = jnp.sum(codebook ** 2, axis=1).reshape(1, K)
    iota_f = jnp.arange(K, dtype=jnp.float32).reshape(1, K)

    idx = pl.pallas_call(
        _dist_body,
        grid=(NTB,),
        in_specs=[
            pl.BlockSpec((1, TB, D), lambda i: (i, 0, 0)),
            pl.BlockSpec((1, K), lambda i: (0, 0)),
            pl.BlockSpec((K, D), lambda i: (0, 0)),
            pl.BlockSpec((1, K), lambda i: (0, 0)),
        ],
        out_specs=pl.BlockSpec((1, 1, TB), lambda i: (i, 0, 0)),
        out_shape=jax.ShapeDtypeStruct((NTB, 1, TB), jnp.int32),
        compiler_params=pltpu.CompilerParams(vmem_limit_bytes=100 * 1024 * 1024),
    )(flat, cnorm, codebook, iota_f)

    q = _sc_gather(codebook, idx.reshape(N // _CHUNK, _CHUNK))

    outT, mse_rows = pl.pallas_call(
        _combine_body,
        grid=(NTB,),
        in_specs=[
            pl.BlockSpec((1, TB, D), lambda i: (i, 0, 0)),
            pl.BlockSpec((1, TB, _PD), lambda i: (i, 0, 0)),
        ],
        out_specs=[
            pl.BlockSpec((1, D, TB), lambda i: (i // (HW // TB), 0, i % (HW // TB))),
            pl.BlockSpec((1, 1, TB), lambda i: (i, 0, 0)),
        ],
        out_shape=[
            jax.ShapeDtypeStruct((B, D, HW), jnp.float32),
            jax.ShapeDtypeStruct((NTB, 1, TB), jnp.float32),
        ],
    )(flat, q.reshape(NTB, TB, _PD))

    mse = jnp.sum(mse_rows) / (N * D)
    kld = jnp.sum(kld_rows) / N
    vq_loss = (mse + kld) * BETA + mse
    return (outT.reshape(B, D, 32, 32), vq_loss)
